# baseline (speedup 1.0000x reference)
import functools
import jax
import jax.numpy as jnp
from jax import lax
from jax.experimental import pallas as pl
from jax.experimental.pallas import tpu as pltpu
from jax.experimental.pallas import tpu_sc as plsc

N_NODE = 1000000
N_DIM = 64
BATCH = 16384

NC = 2
NS = 16
NW = NC * NS
B_PER_W = BATCH // NW
GCHUNK = 128
NG = B_PER_W // GCHUNK


def _line_score_kernel(u_hbm, v_hbm, node_hbm, ctx_hbm, out_hbm,
                       idx_u, idx_v, rows_u, rows_v, tbuf, out_v, sem):
    wid = lax.axis_index("s") * NC + lax.axis_index("c")
    base = wid * B_PER_W

    pltpu.sync_copy(u_hbm.at[pl.ds(base, B_PER_W)], idx_u)
    pltpu.sync_copy(v_hbm.at[pl.ds(base, B_PER_W)], idx_v)

    copies = []
    for g in range(NG):
        sl = pl.ds(g * GCHUNK, GCHUNK)
        copies.append(pltpu.async_copy(node_hbm.at[idx_u.at[sl]], rows_u.at[sl], sem))
        copies.append(pltpu.async_copy(ctx_hbm.at[idx_v.at[sl]], rows_v.at[sl], sem))
    for c in copies:
        c.wait()

    lanes = lax.iota(jnp.int32, 16)

    def body(rb, _):
        row0 = rb * 16
        acc = jnp.zeros((16,), jnp.float32)
        for i in range(16):
            row = row0 + i
            t = rows_u[row, pl.ds(0, 16)] * rows_v[row, pl.ds(0, 16)]
            for c in range(1, N_DIM // 16):
                t += rows_u[row, pl.ds(c * 16, 16)] * rows_v[row, pl.ds(c * 16, 16)]
            acc = jnp.where(lanes == i, jnp.sum(t), acc)
        out_v[pl.ds(row0, 16)] = acc
        return ()

    lax.fori_loop(0, B_PER_W // 16, body, ())

    pltpu.sync_copy(out_v, out_hbm.at[pl.ds(base, B_PER_W)])


@jax.jit
def kernel(u, v, node_embed, context_node_embed):
    k = functools.partial(
        pl.kernel,
        out_type=jax.ShapeDtypeStruct((BATCH,), jnp.float32),
        mesh=plsc.VectorSubcoreMesh(core_axis_name="c", subcore_axis_name="s"),
        compiler_params=pltpu.CompilerParams(
            needs_layout_passes=False, use_tc_tiling_on_sc=False),
        scratch_types=[
            pltpu.VMEM((B_PER_W,), jnp.int32),
            pltpu.VMEM((B_PER_W,), jnp.int32),
            pltpu.VMEM((B_PER_W, N_DIM), jnp.float32),
            pltpu.VMEM((B_PER_W, N_DIM), jnp.float32),
            pltpu.VMEM((256,), jnp.float32),
            pltpu.VMEM((B_PER_W,), jnp.float32),
            pltpu.SemaphoreType.DMA,
        ],
    )(_line_score_kernel)
    return k(u, v, node_embed, context_node_embed)


# speedup vs baseline: 1.5608x; 1.5608x over previous
import functools
import jax
import jax.numpy as jnp
from jax import lax
from jax.experimental import pallas as pl
from jax.experimental.pallas import tpu as pltpu
from jax.experimental.pallas import tpu_sc as plsc

N_NODE = 1000000
N_DIM = 64
BATCH = 16384

NC = 2
NS = 16
NW = NC * NS
B_PER_W = BATCH // NW
CHUNK = 32
NCHUNK = B_PER_W // CHUNK


def _line_score_kernel(u_hbm, v_hbm, node_hbm, ctx_hbm, out_hbm,
                       idx_u, idx_v, rows_u, rows_v, out_v, sem):
    wid = lax.axis_index("s") * NC + lax.axis_index("c")
    base = wid * B_PER_W

    pltpu.sync_copy(u_hbm.at[pl.ds(base, B_PER_W)], idx_u)
    pltpu.sync_copy(v_hbm.at[pl.ds(base, B_PER_W)], idx_v)

    lanes = lax.iota(jnp.int32, 16)

    def chunk_body(ci, _):
        coff = ci * CHUNK
        descs = []
        for sb in range(CHUNK // 16):
            nu16 = idx_u[pl.ds(coff + sb * 16, 16)]
            nv16 = idx_v[pl.ds(coff + sb * 16, 16)]
            for i in range(16):
                e = sb * 16 + i
                descs.append(pltpu.async_copy(node_hbm.at[nu16[i]], rows_u.at[e], sem))
                descs.append(pltpu.async_copy(ctx_hbm.at[nv16[i]], rows_v.at[e], sem))
        for d in descs:
            d.wait()
        for sb in range(CHUNK // 16):
            acc = jnp.zeros((16,), jnp.float32)
            for i in range(16):
                e = sb * 16 + i
                t = rows_u[e, pl.ds(0, 16)] * rows_v[e, pl.ds(0, 16)]
                for c in range(1, N_DIM // 16):
                    t += rows_u[e, pl.ds(c * 16, 16)] * rows_v[e, pl.ds(c * 16, 16)]
                acc = jnp.where(lanes == i, jnp.sum(t), acc)
            out_v[pl.ds(coff + sb * 16, 16)] = acc
        return ()

    lax.fori_loop(0, NCHUNK, chunk_body, ())

    pltpu.sync_copy(out_v, out_hbm.at[pl.ds(base, B_PER_W)])


@jax.jit
def kernel(u, v, node_embed, context_node_embed):
    k = functools.partial(
        pl.kernel,
        out_type=jax.ShapeDtypeStruct((BATCH,), jnp.float32),
        mesh=plsc.VectorSubcoreMesh(core_axis_name="c", subcore_axis_name="s"),
        compiler_params=pltpu.CompilerParams(needs_layout_passes=False),
        scratch_types=[
            pltpu.VMEM((B_PER_W,), jnp.int32),
            pltpu.VMEM((B_PER_W,), jnp.int32),
            pltpu.VMEM((CHUNK, N_DIM), jnp.float32),
            pltpu.VMEM((CHUNK, N_DIM), jnp.float32),
            pltpu.VMEM((B_PER_W,), jnp.float32),
            pltpu.SemaphoreType.DMA,
        ],
    )(_line_score_kernel)
    return k(u, v, node_embed, context_node_embed)


# speedup vs baseline: 1.9387x; 1.2421x over previous
import functools
import jax
import jax.numpy as jnp
from jax import lax
from jax.experimental import pallas as pl
from jax.experimental.pallas import tpu as pltpu
from jax.experimental.pallas import tpu_sc as plsc

N_NODE = 1000000
N_DIM = 64
BATCH = 16384

NC = 2
NS = 16
NW = NC * NS
B_PER_W = BATCH // NW
CHUNK = 2
NCHUNK = B_PER_W // CHUNK


def _line_score_kernel(u_hbm, v_hbm, node_hbm, ctx_hbm, out_hbm,
                       idx_u, idx_v, blk_u, blk_v, out_v, sem):
    wid = lax.axis_index("s") * NC + lax.axis_index("c")
    base = wid * B_PER_W

    pltpu.sync_copy(u_hbm.at[pl.ds(base, B_PER_W)], idx_u.at[pl.ds(0, B_PER_W)])
    pltpu.sync_copy(v_hbm.at[pl.ds(base, B_PER_W)], idx_v.at[pl.ds(0, B_PER_W)])

    lanes = lax.iota(jnp.int32, 16)

    def chunk_body(ci, acc):
        coff = ci * CHUNK
        nu = idx_u[pl.ds(coff, 16)]
        nv = idx_v[pl.ds(coff, 16)]
        descs = []
        for e in range(CHUNK):
            bu = pl.multiple_of(lax.shift_left(lax.shift_right_logical(nu[e], 7), 7), 128)
            bv = pl.multiple_of(lax.shift_left(lax.shift_right_logical(nv[e], 7), 7), 128)
            descs.append(pltpu.async_copy(
                node_hbm.at[pl.ds(0, N_DIM), pl.ds(bu, 128)], blk_u.at[e], sem))
            descs.append(pltpu.async_copy(
                ctx_hbm.at[pl.ds(0, N_DIM), pl.ds(bv, 128)], blk_v.at[e], sem))
        for d in descs:
            d.wait()
        for e in range(CHUNK):
            lu = jnp.bitwise_and(nu[e], 127)
            lv = jnp.bitwise_and(nv[e], 127)
            t = jnp.zeros((16,), jnp.float32)
            for c in range(N_DIM // 16):
                d16 = lanes + (c * 16)
                gu = plsc.load_gather(blk_u.at[e], [d16, jnp.full((16,), lu, jnp.int32)])
                gv = plsc.load_gather(blk_v.at[e], [d16, jnp.full((16,), lv, jnp.int32)])
                t += gu * gv
            lane = lax.rem(coff + e, 16)
            acc = jnp.where(lanes == lane, jnp.sum(t), acc)
        @pl.when(lax.rem(coff + CHUNK, 16) == 0)
        def _store():
            out_v[pl.ds(pl.multiple_of(coff + CHUNK - 16, 16), 16)] = acc
        return acc

    lax.fori_loop(0, NCHUNK, chunk_body, jnp.zeros((16,), jnp.float32))

    pltpu.sync_copy(out_v, out_hbm.at[pl.ds(base, B_PER_W)])


@jax.jit
def kernel(u, v, node_embed, context_node_embed):
    k = functools.partial(
        pl.kernel,
        out_type=jax.ShapeDtypeStruct((BATCH,), jnp.float32),
        mesh=plsc.VectorSubcoreMesh(core_axis_name="c", subcore_axis_name="s"),
        compiler_params=pltpu.CompilerParams(needs_layout_passes=False),
        scratch_types=[
            pltpu.VMEM((B_PER_W + 16,), jnp.int32),
            pltpu.VMEM((B_PER_W + 16,), jnp.int32),
            pltpu.VMEM((CHUNK, N_DIM, 128), jnp.float32),
            pltpu.VMEM((CHUNK, N_DIM, 128), jnp.float32),
            pltpu.VMEM((B_PER_W,), jnp.float32),
            pltpu.SemaphoreType.DMA,
        ],
    )(_line_score_kernel)
    return k(u, v, node_embed.T, context_node_embed.T)


# speedup vs baseline: 2.7434x; 1.4151x over previous
import functools
import jax
import jax.numpy as jnp
from jax import lax
from jax.experimental import pallas as pl
from jax.experimental.pallas import tpu as pltpu
from jax.experimental.pallas import tpu_sc as plsc

N_NODE = 1000000
N_DIM = 64
BATCH = 16384

NC = 2
NS = 16
NW = NC * NS
B_PER_W = BATCH // NW
WBLK = 2
WNODES = WBLK * 128
NBLOCKS = (N_NODE + 127) // 128
MAX_WSTART = NBLOCKS - WBLK
GCHUNK = 128


def _extract_one_table(tbl_hbm, nodes, winA, winB, stage, sem):
    lanes = lax.iota(jnp.int32, 16)

    def win_src(ws):
        wc = jnp.minimum(ws, MAX_WSTART)
        off = pl.multiple_of(wc * 128, 128)
        return wc, tbl_hbm.at[pl.ds(0, N_DIM), pl.ds(off, WNODES)]

    def node_at(cur):
        return nodes[pl.ds(cur, 16)][0]

    def extract(buf, ws, cursor):
        wc = jnp.minimum(ws, MAX_WSTART)
        wend = (wc + WBLK) * 128

        def cond(cur):
            return jnp.logical_and(cur < B_PER_W, node_at(cur) < wend)

        def body(cur):
            nl = node_at(cur) - wc * 128
            nlv = jnp.full((16,), nl, jnp.int32)
            for c in range(N_DIM // 16):
                g = plsc.load_gather(buf, [lanes + c * 16, nlv])
                stage[cur, pl.ds(c * 16, 16)] = g
            return cur + 1

        return lax.while_loop(cond, body, cursor)

    first = lax.shift_right_logical(node_at(0), 7)
    wc0, src0 = win_src(first)
    dA = pltpu.async_copy(src0, winA, sem)

    def outer_cond(state):
        ws, cursor = state
        return cursor < B_PER_W

    def outer_body(state):
        ws, cursor = state
        pltpu.make_async_copy(win_src(ws)[1], winA, sem).wait()
        _, srcB = win_src(ws + WBLK)
        pltpu.async_copy(srcB, winB, sem)
        cursor = extract(winA, ws, cursor)
        pltpu.make_async_copy(srcB, winB, sem).wait()
        _, srcA = win_src(ws + 2 * WBLK)
        pltpu.async_copy(srcA, winA, sem)
        cursor = extract(winB, ws + WBLK, cursor)
        return (ws + 2 * WBLK, cursor)

    ws_end, _ = lax.while_loop(outer_cond, outer_body, (first, jnp.int32(0)))
    pltpu.make_async_copy(win_src(ws_end)[1], winA, sem).wait()


def _extract_kernel(us_hbm, vs_hbm, node_hbm, ctx_hbm, eu_hbm, ev_hbm,
                    nodes, winA, winB, stage, sem):
    wid = lax.axis_index("s") * NC + lax.axis_index("c")
    base = wid * B_PER_W

    pltpu.sync_copy(us_hbm.at[pl.ds(base, B_PER_W)], nodes.at[pl.ds(0, B_PER_W)])
    _extract_one_table(node_hbm, nodes, winA, winB, stage, sem)
    pltpu.sync_copy(stage, eu_hbm.at[pl.ds(base, B_PER_W)])

    pltpu.sync_copy(vs_hbm.at[pl.ds(base, B_PER_W)], nodes.at[pl.ds(0, B_PER_W)])
    _extract_one_table(ctx_hbm, nodes, winA, winB, stage, sem)
    pltpu.sync_copy(stage, ev_hbm.at[pl.ds(base, B_PER_W)])


def _dot_kernel(iu_hbm, iv_hbm, eu_hbm, ev_hbm, out_hbm,
                idx_u, idx_v, rows_u, rows_v, out_v, sem):
    wid = lax.axis_index("s") * NC + lax.axis_index("c")
    base = wid * B_PER_W

    pltpu.sync_copy(iu_hbm.at[pl.ds(base, B_PER_W)], idx_u)
    pltpu.sync_copy(iv_hbm.at[pl.ds(base, B_PER_W)], idx_v)

    lanes = lax.iota(jnp.int32, 16)

    def chunk_body(ci, _):
        coff = ci * GCHUNK
        cu = pltpu.async_copy(eu_hbm.at[idx_u.at[pl.ds(coff, GCHUNK)]], rows_u, sem)
        cv = pltpu.async_copy(ev_hbm.at[idx_v.at[pl.ds(coff, GCHUNK)]], rows_v, sem)
        cu.wait()
        cv.wait()
        for sb in range(GCHUNK // 16):
            acc = jnp.zeros((16,), jnp.float32)
            for i in range(16):
                e = sb * 16 + i
                t = rows_u[e, pl.ds(0, 16)] * rows_v[e, pl.ds(0, 16)]
                for c in range(1, N_DIM // 16):
                    t += rows_u[e, pl.ds(c * 16, 16)] * rows_v[e, pl.ds(c * 16, 16)]
                acc = jnp.where(lanes == i, jnp.sum(t), acc)
            out_v[pl.ds(coff + sb * 16, 16)] = acc
        return ()

    lax.fori_loop(0, B_PER_W // GCHUNK, chunk_body, ())

    pltpu.sync_copy(out_v, out_hbm.at[pl.ds(base, B_PER_W)])


@jax.jit
def kernel(u, v, node_embed, context_node_embed):
    iota = lax.iota(jnp.int32, BATCH)
    u_s, pu = lax.sort_key_val(u, iota)
    v_s, pv = lax.sort_key_val(v, iota)
    inv_pu = jnp.zeros((BATCH,), jnp.int32).at[pu].set(iota)
    inv_pv = jnp.zeros((BATCH,), jnp.int32).at[pv].set(iota)

    mesh = plsc.VectorSubcoreMesh(core_axis_name="c", subcore_axis_name="s")
    params = pltpu.CompilerParams(needs_layout_passes=False)

    extract = functools.partial(
        pl.kernel,
        out_type=(jax.ShapeDtypeStruct((BATCH, 128), jnp.float32),
                  jax.ShapeDtypeStruct((BATCH, 128), jnp.float32)),
        mesh=mesh,
        compiler_params=params,
        scratch_types=[
            pltpu.VMEM((B_PER_W + 16,), jnp.int32),
            pltpu.VMEM((N_DIM, WNODES), jnp.float32),
            pltpu.VMEM((N_DIM, WNODES), jnp.float32),
            pltpu.VMEM((B_PER_W, 128), jnp.float32),
            pltpu.SemaphoreType.DMA,
        ],
    )(_extract_kernel)
    eu, ev = extract(u_s, v_s, node_embed.T, context_node_embed.T)

    dot = functools.partial(
        pl.kernel,
        out_type=jax.ShapeDtypeStruct((BATCH,), jnp.float32),
        mesh=mesh,
        compiler_params=params,
        scratch_types=[
            pltpu.VMEM((B_PER_W,), jnp.int32),
            pltpu.VMEM((B_PER_W,), jnp.int32),
            pltpu.VMEM((GCHUNK, 128), jnp.float32),
            pltpu.VMEM((GCHUNK, 128), jnp.float32),
            pltpu.VMEM((B_PER_W,), jnp.float32),
            pltpu.SemaphoreType.DMA,
        ],
    )(_dot_kernel)
    return dot(inv_pu, inv_pv, eu, ev)


# speedup vs baseline: 4.3992x; 1.6036x over previous
import functools
import jax
import jax.numpy as jnp
from jax import lax
from jax.experimental import pallas as pl
from jax.experimental.pallas import tpu as pltpu
from jax.experimental.pallas import tpu_sc as plsc

N_NODE = 1000000
N_DIM = 64
BATCH = 16384

NC = 2
NS = 16
NW = NC * NS
B_PER_W = BATCH // NW
WBLK = 1
NBUF = 4
WNODES = WBLK * 128
NBLOCKS = (N_NODE + 127) // 128
MAX_WSTART = NBLOCKS - WBLK
GCHUNK = 128


def _extract_one_table(tbl_hbm, nodes, winA, stage, sem):
    lanes = lax.iota(jnp.int32, 16)

    def win_src(ws):
        wc = jnp.minimum(ws, MAX_WSTART)
        off = pl.multiple_of(wc * 128, 128)
        return wc, tbl_hbm.at[pl.ds(0, N_DIM), pl.ds(off, WNODES)]

    def node_at(cur):
        return nodes[pl.ds(cur, 16)][0]

    def extract(buf, ws, cursor):
        wc = jnp.minimum(ws, MAX_WSTART)
        wend = (wc + WBLK) * 128

        def cond(cur):
            return jnp.logical_and(cur < B_PER_W, node_at(cur) < wend)

        def body(cur):
            nl = node_at(cur) - wc * 128
            nlv = jnp.full((16,), nl, jnp.int32)
            for c in range(N_DIM // 16):
                g = plsc.load_gather(buf, [lanes + c * 16, nlv])
                stage[cur, pl.ds(c * 16, 16)] = g
            return cur + 1

        return lax.while_loop(cond, body, cursor)

    first = lax.shift_right_logical(node_at(0), 7)
    bufs = [winA.at[b] for b in range(NBUF)]
    for b in range(NBUF):
        pltpu.async_copy(win_src(first + b)[1], bufs[b], sem)

    def outer_cond(state):
        ws, cursor = state
        return cursor < B_PER_W

    def outer_body(state):
        ws, cursor = state
        for b in range(NBUF):
            pltpu.make_async_copy(win_src(ws + b)[1], bufs[b], sem).wait()
            cursor = extract(bufs[b], ws + b, cursor)
            pltpu.async_copy(win_src(ws + NBUF + b)[1], bufs[b], sem)
        return (ws + NBUF, cursor)

    ws_end, _ = lax.while_loop(outer_cond, outer_body, (first, jnp.int32(0)))
    for b in range(NBUF):
        pltpu.make_async_copy(win_src(ws_end + b)[1], bufs[b], sem).wait()


def _extract_kernel(us_hbm, vs_hbm, node_hbm, ctx_hbm, eu_hbm, ev_hbm,
                    nodes, winA, stage, sem):
    wid = lax.axis_index("s") * NC + lax.axis_index("c")
    base = wid * B_PER_W

    pltpu.sync_copy(us_hbm.at[pl.ds(base, B_PER_W)], nodes.at[pl.ds(0, B_PER_W)])
    _extract_one_table(node_hbm, nodes, winA, stage, sem)
    pltpu.sync_copy(stage, eu_hbm.at[pl.ds(base, B_PER_W)])

    pltpu.sync_copy(vs_hbm.at[pl.ds(base, B_PER_W)], nodes.at[pl.ds(0, B_PER_W)])
    _extract_one_table(ctx_hbm, nodes, winA, stage, sem)
    pltpu.sync_copy(stage, ev_hbm.at[pl.ds(base, B_PER_W)])


def _dot_kernel(iu_hbm, iv_hbm, eu_hbm, ev_hbm, out_hbm,
                idx_u, idx_v, rows_u, rows_v, out_v, sem):
    wid = lax.axis_index("s") * NC + lax.axis_index("c")
    base = wid * B_PER_W

    pltpu.sync_copy(iu_hbm.at[pl.ds(base, B_PER_W)], idx_u)
    pltpu.sync_copy(iv_hbm.at[pl.ds(base, B_PER_W)], idx_v)

    lanes = lax.iota(jnp.int32, 16)

    def chunk_body(ci, _):
        coff = ci * GCHUNK
        cu = pltpu.async_copy(eu_hbm.at[idx_u.at[pl.ds(coff, GCHUNK)]], rows_u, sem)
        cv = pltpu.async_copy(ev_hbm.at[idx_v.at[pl.ds(coff, GCHUNK)]], rows_v, sem)
        cu.wait()
        cv.wait()
        for sb in range(GCHUNK // 16):
            acc = jnp.zeros((16,), jnp.float32)
            for i in range(16):
                e = sb * 16 + i
                t = rows_u[e, pl.ds(0, 16)] * rows_v[e, pl.ds(0, 16)]
                for c in range(1, N_DIM // 16):
                    t += rows_u[e, pl.ds(c * 16, 16)] * rows_v[e, pl.ds(c * 16, 16)]
                acc = jnp.where(lanes == i, jnp.sum(t), acc)
            out_v[pl.ds(coff + sb * 16, 16)] = acc
        return ()

    lax.fori_loop(0, B_PER_W // GCHUNK, chunk_body, ())

    pltpu.sync_copy(out_v, out_hbm.at[pl.ds(base, B_PER_W)])


@jax.jit
def kernel(u, v, node_embed, context_node_embed):
    iota = lax.iota(jnp.int32, BATCH)
    u_s, pu = lax.sort_key_val(u, iota)
    v_s, pv = lax.sort_key_val(v, iota)
    inv_pu = jnp.zeros((BATCH,), jnp.int32).at[pu].set(iota)
    inv_pv = jnp.zeros((BATCH,), jnp.int32).at[pv].set(iota)

    mesh = plsc.VectorSubcoreMesh(core_axis_name="c", subcore_axis_name="s")
    params = pltpu.CompilerParams(needs_layout_passes=False)

    extract = functools.partial(
        pl.kernel,
        out_type=(jax.ShapeDtypeStruct((BATCH, 128), jnp.float32),
                  jax.ShapeDtypeStruct((BATCH, 128), jnp.float32)),
        mesh=mesh,
        compiler_params=params,
        scratch_types=[
            pltpu.VMEM((B_PER_W + 16,), jnp.int32),
            pltpu.VMEM((NBUF, N_DIM, WNODES), jnp.float32),
            pltpu.VMEM((B_PER_W, 128), jnp.float32),
            pltpu.SemaphoreType.DMA,
        ],
    )(_extract_kernel)
    eu, ev = extract(u_s, v_s, node_embed.T, context_node_embed.T)

    dot = functools.partial(
        pl.kernel,
        out_type=jax.ShapeDtypeStruct((BATCH,), jnp.float32),
        mesh=mesh,
        compiler_params=params,
        scratch_types=[
            pltpu.VMEM((B_PER_W,), jnp.int32),
            pltpu.VMEM((B_PER_W,), jnp.int32),
            pltpu.VMEM((GCHUNK, 128), jnp.float32),
            pltpu.VMEM((GCHUNK, 128), jnp.float32),
            pltpu.VMEM((B_PER_W,), jnp.float32),
            pltpu.SemaphoreType.DMA,
        ],
    )(_dot_kernel)
    return dot(inv_pu, inv_pv, eu, ev)


# speedup vs baseline: 4.7689x; 1.0840x over previous
import functools
import jax
import jax.numpy as jnp
from jax import lax
from jax.experimental import pallas as pl
from jax.experimental.pallas import tpu as pltpu
from jax.experimental.pallas import tpu_sc as plsc

N_NODE = 1000000
N_DIM = 64
BATCH = 16384

NC = 2
NS = 16
NW = NC * NS
B_PER_W = BATCH // NW
WBLK = 1
NBUF = 6
WNODES = WBLK * 128
NBLOCKS = (N_NODE + 127) // 128
MAX_WSTART = NBLOCKS - WBLK
GCHUNK = 128


def _extract_one_table(tbl_hbm, nodes, winA, stage, sem):
    lanes = lax.iota(jnp.int32, 16)

    def win_src(ws):
        wc = jnp.minimum(ws, MAX_WSTART)
        off = pl.multiple_of(wc * 128, 128)
        return wc, tbl_hbm.at[pl.ds(0, N_DIM), pl.ds(off, WNODES)]

    def node_at(cur):
        return nodes[pl.ds(cur, 16)][0]

    def extract(buf, ws, cursor):
        wc = jnp.minimum(ws, MAX_WSTART)
        wend = (wc + WBLK) * 128

        def cond(cur):
            return jnp.logical_and(cur < B_PER_W, node_at(cur) < wend)

        def body(cur):
            nl = node_at(cur) - wc * 128
            nlv = jnp.full((16,), nl, jnp.int32)
            for c in range(N_DIM // 16):
                g = plsc.load_gather(buf, [lanes + c * 16, nlv])
                stage[cur, pl.ds(c * 16, 16)] = g
            return cur + 1

        return lax.while_loop(cond, body, cursor)

    first = lax.shift_right_logical(node_at(0), 7)
    bufs = [winA.at[b] for b in range(NBUF)]
    for b in range(NBUF):
        pltpu.async_copy(win_src(first + b)[1], bufs[b], sem)

    def outer_cond(state):
        ws, cursor = state
        return cursor < B_PER_W

    def outer_body(state):
        ws, cursor = state
        for b in range(NBUF):
            pltpu.make_async_copy(win_src(ws + b)[1], bufs[b], sem).wait()
            cursor = extract(bufs[b], ws + b, cursor)
            pltpu.async_copy(win_src(ws + NBUF + b)[1], bufs[b], sem)
        return (ws + NBUF, cursor)

    ws_end, _ = lax.while_loop(outer_cond, outer_body, (first, jnp.int32(0)))
    for b in range(NBUF):
        pltpu.make_async_copy(win_src(ws_end + b)[1], bufs[b], sem).wait()


def _extract_kernel(us_hbm, vs_hbm, node_hbm, ctx_hbm, eu_hbm, ev_hbm,
                    nodes, winA, stage, sem):
    wid = lax.axis_index("s") * NC + lax.axis_index("c")
    base = wid * B_PER_W

    pltpu.sync_copy(us_hbm.at[pl.ds(base, B_PER_W)], nodes.at[pl.ds(0, B_PER_W)])
    _extract_one_table(node_hbm, nodes, winA, stage, sem)
    pltpu.sync_copy(stage, eu_hbm.at[pl.ds(base, B_PER_W)])

    pltpu.sync_copy(vs_hbm.at[pl.ds(base, B_PER_W)], nodes.at[pl.ds(0, B_PER_W)])
    _extract_one_table(ctx_hbm, nodes, winA, stage, sem)
    pltpu.sync_copy(stage, ev_hbm.at[pl.ds(base, B_PER_W)])


def _dot_kernel(iu_hbm, iv_hbm, eu_hbm, ev_hbm, out_hbm,
                idx_u, idx_v, rows_u, rows_v, out_v, sem):
    wid = lax.axis_index("s") * NC + lax.axis_index("c")
    base = wid * B_PER_W

    pltpu.sync_copy(iu_hbm.at[pl.ds(base, B_PER_W)], idx_u)
    pltpu.sync_copy(iv_hbm.at[pl.ds(base, B_PER_W)], idx_v)

    lanes = lax.iota(jnp.int32, 16)

    def chunk_body(ci, _):
        coff = ci * GCHUNK
        cu = pltpu.async_copy(eu_hbm.at[idx_u.at[pl.ds(coff, GCHUNK)]], rows_u, sem)
        cv = pltpu.async_copy(ev_hbm.at[idx_v.at[pl.ds(coff, GCHUNK)]], rows_v, sem)
        cu.wait()
        cv.wait()
        for sb in range(GCHUNK // 16):
            acc = jnp.zeros((16,), jnp.float32)
            for i in range(16):
                e = sb * 16 + i
                t = rows_u[e, pl.ds(0, 16)] * rows_v[e, pl.ds(0, 16)]
                for c in range(1, N_DIM // 16):
                    t += rows_u[e, pl.ds(c * 16, 16)] * rows_v[e, pl.ds(c * 16, 16)]
                acc = jnp.where(lanes == i, jnp.sum(t), acc)
            out_v[pl.ds(coff + sb * 16, 16)] = acc
        return ()

    lax.fori_loop(0, B_PER_W // GCHUNK, chunk_body, ())

    pltpu.sync_copy(out_v, out_hbm.at[pl.ds(base, B_PER_W)])


@jax.jit
def kernel(u, v, node_embed, context_node_embed):
    iota = lax.iota(jnp.int32, BATCH)
    u_s, pu = lax.sort_key_val(u, iota)
    v_s, pv = lax.sort_key_val(v, iota)
    inv_pu = jnp.zeros((BATCH,), jnp.int32).at[pu].set(iota)
    inv_pv = jnp.zeros((BATCH,), jnp.int32).at[pv].set(iota)

    mesh = plsc.VectorSubcoreMesh(core_axis_name="c", subcore_axis_name="s")
    params = pltpu.CompilerParams(needs_layout_passes=False)

    extract = functools.partial(
        pl.kernel,
        out_type=(jax.ShapeDtypeStruct((BATCH, 128), jnp.float32),
                  jax.ShapeDtypeStruct((BATCH, 128), jnp.float32)),
        mesh=mesh,
        compiler_params=params,
        scratch_types=[
            pltpu.VMEM((B_PER_W + 16,), jnp.int32),
            pltpu.VMEM((NBUF, N_DIM, WNODES), jnp.float32),
            pltpu.VMEM((B_PER_W, 128), jnp.float32),
            pltpu.SemaphoreType.DMA,
        ],
    )(_extract_kernel)
    eu, ev = extract(u_s, v_s, node_embed.T, context_node_embed.T)

    dot = functools.partial(
        pl.kernel,
        out_type=jax.ShapeDtypeStruct((BATCH,), jnp.float32),
        mesh=mesh,
        compiler_params=params,
        scratch_types=[
            pltpu.VMEM((B_PER_W,), jnp.int32),
            pltpu.VMEM((B_PER_W,), jnp.int32),
            pltpu.VMEM((GCHUNK, 128), jnp.float32),
            pltpu.VMEM((GCHUNK, 128), jnp.float32),
            pltpu.VMEM((B_PER_W,), jnp.float32),
            pltpu.SemaphoreType.DMA,
        ],
    )(_dot_kernel)
    return dot(inv_pu, inv_pv, eu, ev)
